# baseline (speedup 1.0000x reference)
import jax
import jax.numpy as jnp
from jax.experimental import pallas as pl
from jax.experimental.pallas import tpu as pltpu

_T = 16
_BLK = 256
_N = 16


def _body(x_ref, out_ref, mn_ref, mx_ref):
    i = pl.program_id(0)

    @pl.when(i < _N)
    def _reduce():
        blk = x_ref[...]
        bmin = jnp.min(blk)
        bmax = jnp.max(blk)

        @pl.when(i == 0)
        def _init():
            mn_ref[0] = bmin
            mx_ref[0] = bmax

        @pl.when(i > 0)
        def _acc():
            mn_ref[0] = jnp.minimum(mn_ref[0], bmin)
            mx_ref[0] = jnp.maximum(mx_ref[0], bmax)

    @pl.when(i >= _N)
    def _encode():
        mn = mn_ref[0]
        mx = mx_ref[0]
        xblk = x_ref[...]
        xn = jnp.clip((xblk - mn) / (mx - mn + 1e-8), 0.0, 1.0)
        lat = ((1.0 - xn) * (_T - 1)).astype(jnp.int32)
        t = jax.lax.broadcasted_iota(jnp.int32, (_BLK, _T, xblk.shape[1]), 1)
        out_ref[...] = (lat[:, None, :] == t).astype(jnp.float32)


def kernel(x):
    B, F = x.shape
    return pl.pallas_call(
        _body,
        grid=(2 * _N,),
        in_specs=(pl.BlockSpec((_BLK, F), lambda i: (jnp.where(i < _N, i, i - _N), 0)),),
        out_specs=pl.BlockSpec((_BLK, _T, F), lambda i: (jnp.maximum(i - _N, 0), 0, 0)),
        out_shape=jax.ShapeDtypeStruct((B, _T, F), jnp.float32),
        scratch_shapes=[
            pltpu.SMEM((1,), jnp.float32),
            pltpu.SMEM((1,), jnp.float32),
        ],
        compiler_params=pltpu.CompilerParams(
            dimension_semantics=("arbitrary",),
        ),
    )(x)
